# baseline (speedup 1.0000x reference)
import jax
import jax.numpy as jnp
from jax import lax
from jax.experimental import pallas as pl
from jax.experimental.pallas import tpu as pltpu
from jax.experimental.pallas import tpu_sc as plsc

B, N, L = 4096, 50, 5
D = 64
G = B * N
NC, NS = 2, 16
NW = NC * NS
GW = G // NW
CG = 128
NCH = GW // CG


def _sc_body(uidx, iidx, ust, uvt, ist, ivt,
             us_out, uv_out, is_out, iv_out,
             nat_u, nat_i, rows_u, rows_i, srows_u, srows_i,
             acc_u, acc_i, sacc_sp_u, sacc_sp_i, zeros_v, gidx2,
             gsem_u, gsem_i, isem_u, isem_i, wbsem_u, wbsem_i, ssem):
    sid = lax.axis_index("s")
    wid = sid * NC + lax.axis_index("c")
    base = wid * GW

    FU = (uidx, ust, uvt, us_out, uv_out, nat_u, rows_u, srows_u,
          acc_u, sacc_sp_u, gsem_u, isem_u, wbsem_u)
    FI = (iidx, ist, ivt, is_out, iv_out, nat_i, rows_i, srows_i,
          acc_i, sacc_sp_i, gsem_i, isem_i, wbsem_i)

    for t in range(CG // 16):
        zeros_v[pl.ds(t * 16, 16)] = jnp.zeros((16,), jnp.float32)
    for j in range(L):
        for t in range(CG // 16):
            gidx2[j, pl.ds(t * 16, 16)] = sid * CG + (
                ((lax.iota(jnp.int32, 16) + (j * CG + t * 16)) * 6554) >> 15)

    def fire_idx(f, c):
        idx_hbm, nat_v, isem = f[0], f[5], f[11]
        pltpu.async_copy(idx_hbm.at[pl.ds((base + c * CG) * L, CG * L)],
                         nat_v, isem)

    def wait_idx(f):
        idx_hbm, nat_v, isem = f[0], f[5], f[11]
        pltpu.make_async_copy(idx_hbm.at[pl.ds(0, CG * L)], nat_v,
                              isem).wait()

    def fire_gathers(f):
        st, vt, nat_v, rows_v, srows_v, gsem = (
            f[1], f[2], f[5], f[6], f[7], f[10])
        for j in range(L):
            sel = nat_v.at[pl.ds(j * CG, CG)]
            pltpu.async_copy(vt.at[sel], rows_v.at[j], gsem)
            pltpu.async_copy(st.at[sel], srows_v.at[j], gsem)

    def wait_gathers(f):
        st, vt, rows_v, srows_v, gsem = f[1], f[2], f[6], f[7], f[10]
        for j in range(L):
            pltpu.make_async_copy(vt.at[pl.ds(0, CG)], rows_v.at[j],
                                  gsem).wait()
            pltpu.make_async_copy(st.at[pl.ds(0, CG)], srows_v.at[j],
                                  gsem).wait()

    def reduce(f):
        rows_v, srows_v, acc_v, sacc_sp = f[6], f[7], f[8], f[9]
        pltpu.sync_copy(zeros_v, sacc_sp.at[pl.ds(sid * CG, CG)])
        hs = [pltpu.async_copy(srows_v.at[j], sacc_sp.at[gidx2.at[j]], ssem,
                               add=True)
              for j in range(L)]

        def body(g, c2):
            p0 = g * L
            accs = [None] * (D // 16)
            for j in range(L):
                pj = (p0 + j) >> 7
                pr = (p0 + j) & (CG - 1)
                for q in range(D // 16):
                    v = rows_v[pj, pr, pl.ds(q * 16, 16)]
                    accs[q] = v if j == 0 else accs[q] + v
            for q in range(D // 16):
                acc_v[g, pl.ds(q * 16, 16)] = accs[q]
            return c2

        lax.fori_loop(0, CG, body, 0, unroll=2)
        for h in hs:
            h.wait()

    def fire_wb(f, c):
        outs, outv, acc_v, sacc_sp, wbsem = f[3], f[4], f[8], f[9], f[12]
        g0 = base + c * CG
        pltpu.async_copy(acc_v, outv.at[pl.ds(g0, CG)], wbsem)
        pltpu.async_copy(sacc_sp.at[pl.ds(sid * CG, CG)],
                         outs.at[pl.ds(g0, CG)], wbsem)

    def wait_wb(f):
        outs, outv, acc_v, sacc_sp, wbsem = f[3], f[4], f[8], f[9], f[12]
        pltpu.make_async_copy(acc_v, outv.at[pl.ds(0, CG)], wbsem).wait()
        pltpu.make_async_copy(sacc_sp.at[pl.ds(sid * CG, CG)],
                              outs.at[pl.ds(0, CG)], wbsem).wait()

    fire_idx(FI, 0)
    pltpu.sync_copy(uidx.at[pl.ds(base * L, CG * L)], nat_u)
    fire_gathers(FU)

    def chunk_body(c, carry):
        wait_gathers(FU)

        @pl.when(c < NCH - 1)
        def _():
            fire_idx(FU, c + 1)

        wait_idx(FI)

        @pl.when(c > 0)
        def _():
            wait_wb(FI)

        fire_gathers(FI)

        @pl.when(c > 0)
        def _():
            wait_wb(FU)

        reduce(FU)
        fire_wb(FU, c)

        wait_gathers(FI)

        @pl.when(c < NCH - 1)
        def _():
            fire_idx(FI, c + 1)
            wait_idx(FU)
            fire_gathers(FU)

        reduce(FI)
        fire_wb(FI, c)
        return carry

    lax.fori_loop(0, NCH, chunk_body, 0)
    wait_wb(FU)
    wait_wb(FI)


def kernel(user_id, item_id, user_scalar_table, user_vector_table,
           item_scalar_table, item_vector_table):
    uidx = user_id.reshape(G * L).astype(jnp.int32)
    iidx = item_id.reshape(G * L).astype(jnp.int32)

    call = pl.kernel(
        _sc_body,
        out_type=(
            jax.ShapeDtypeStruct((G,), jnp.float32),
            jax.ShapeDtypeStruct((G, D), jnp.float32),
            jax.ShapeDtypeStruct((G,), jnp.float32),
            jax.ShapeDtypeStruct((G, D), jnp.float32),
        ),
        mesh=plsc.VectorSubcoreMesh(core_axis_name="c", subcore_axis_name="s"),
        scratch_types=[
            pltpu.VMEM((CG * L,), jnp.int32),
            pltpu.VMEM((CG * L,), jnp.int32),
            pltpu.VMEM((L, CG, D), jnp.float32),
            pltpu.VMEM((L, CG, D), jnp.float32),
            pltpu.VMEM((L, CG), jnp.float32),
            pltpu.VMEM((L, CG), jnp.float32),
            pltpu.VMEM((CG, D), jnp.float32),
            pltpu.VMEM((CG, D), jnp.float32),
            pltpu.VMEM_SHARED((NS * CG,), jnp.float32),
            pltpu.VMEM_SHARED((NS * CG,), jnp.float32),
            pltpu.VMEM((CG,), jnp.float32),
            pltpu.VMEM((L, CG), jnp.int32),
            pltpu.SemaphoreType.DMA,
            pltpu.SemaphoreType.DMA,
            pltpu.SemaphoreType.DMA,
            pltpu.SemaphoreType.DMA,
            pltpu.SemaphoreType.DMA,
            pltpu.SemaphoreType.DMA,
            pltpu.SemaphoreType.DMA,
        ],
        compiler_params=pltpu.CompilerParams(use_tc_tiling_on_sc=False),
    )
    us, uv, is_, iv = call(
        uidx, iidx,
        user_scalar_table.reshape(-1), user_vector_table,
        item_scalar_table.reshape(-1), item_vector_table)
    return (us.reshape(B, N, 1), uv.reshape(B, N, 1, D),
            is_.reshape(B, N, 1), iv.reshape(B, N, 1, D))
